# baseline (speedup 1.0000x reference)
import functools

import jax
import jax.numpy as jnp
from jax import lax
from jax.experimental import pallas as pl
from jax.experimental.pallas import tpu as pltpu

_NUM_CLASSES = 80
_REG_MAX = 16
_NBINS = _REG_MAX + 1
_LANES = 128


def _bw_body(sc_ref, lab_ref, ones_ref, bw_ref):
    s = jax.lax.dot_general(sc_ref[...], ones_ref[...],
                            (((1,), (0,)), ((), ())),
                            preferred_element_type=jnp.float32)
    maskf = (lab_ref[...] != _NUM_CLASSES).astype(jnp.float32)
    bw_ref[...] = s[:, 0:1] * maskf


def _loss_body(pd_ref, bw_ref, x0p_ref, x1p_ref, ap_ref, lab_ref, x0a_ref,
               x1a_ref, ssum_ref, l1_ref, iou_ref, dfl_ref, acc_ref):
    step = pl.program_id(0)
    nsteps = pl.num_programs(0)

    @pl.when(step == 0)
    def _init():
        for i in range(5):
            acc_ref[i] = 0.0

    mask = lab_ref[...] != _NUM_CLASSES
    maskf = mask.astype(jnp.float32)

    x0p = x0p_ref[...]
    x1p = x1p_ref[...]
    x0a = x0a_ref[...]
    x1a = x1a_ref[...]
    l1sum = jnp.sum((jnp.abs(x0p - x0a) + jnp.abs(x1p - x1a)) * maskf)

    inter = jnp.maximum(jnp.minimum(x1p, x1a) - jnp.maximum(x0p, x0a), 0.0)
    union = (x1p - x0p) + (x1a - x0a) - inter
    union_safe = jnp.where(mask, union, 1.0)
    tiou = jnp.where(mask, inter / union_safe, 0.0)
    iousum = jnp.sum(jnp.where(mask, 1.0 - tiou, 0.0))
    npos = jnp.sum(maskf)

    bw = bw_ref[...]
    bwsum = jnp.sum(bw)

    ap = ap_ref[...]
    ltrb_l = jnp.clip(ap - x0a, 0.0, _REG_MAX - 0.01)
    ltrb_r = jnp.clip(x1a - ap, 0.0, _REG_MAX - 0.01)
    pd = pd_ref[...]
    iota3 = lax.broadcasted_iota(jnp.int32, (_NBINS,) + ap.shape, 0)

    def _dfl_half(x, ltrb):
        logS = jnp.log(jnp.sum(jnp.exp(x), axis=0))
        t = ltrb.astype(jnp.int32)
        xt = jnp.sum(jnp.where(iota3 == t[None], x, 0.0), axis=0)
        xt1 = jnp.sum(jnp.where(iota3 == t[None] + 1, x, 0.0), axis=0)
        wl = (t + 1).astype(jnp.float32) - ltrb
        wr = 1.0 - wl
        return (logS - xt) * wl + (logS - xt1) * wr

    dfl = 0.5 * (_dfl_half(pd[:_NBINS], ltrb_l) + _dfl_half(pd[_NBINS:], ltrb_r))
    dflsum = jnp.sum(dfl * bw)

    acc_ref[0] += npos
    acc_ref[1] += l1sum
    acc_ref[2] += iousum
    acc_ref[3] += bwsum
    acc_ref[4] += dflsum

    @pl.when(step == nsteps - 1)
    def _finish():
        np_ = acc_ref[0]
        ssum = ssum_ref[0]
        l1_ref[0] = acc_ref[1] / (np_ * 2.0)
        iou_ref[0] = (acc_ref[2] / np_) * acc_ref[3] / ssum
        dfl_ref[0] = acc_ref[4] / ssum


@functools.partial(jax.jit, static_argnames=("interpret",))
def _run(pred_dist, pred_bboxes, anchor_points, assigned_labels,
         assigned_bboxes, assigned_scores, assigned_scores_sum,
         interpret=False):
    B, L = assigned_labels.shape
    N = B * L
    NR = N // _LANES
    RB = 64
    grid = (NR // RB,)

    C1 = 8192
    sc = assigned_scores.reshape(N, _NUM_CLASSES)
    labn = assigned_labels.reshape(N, 1)
    ones = jnp.ones((_NUM_CLASSES, _LANES), jnp.float32)
    bw_n = pl.pallas_call(
        _bw_body,
        grid=(N // C1,),
        in_specs=[
            pl.BlockSpec((C1, _NUM_CLASSES), lambda i: (i, 0)),
            pl.BlockSpec((C1, 1), lambda i: (i, 0)),
            pl.BlockSpec((_NUM_CLASSES, _LANES), lambda i: (0, 0)),
        ],
        out_specs=pl.BlockSpec((C1, 1), lambda i: (i, 0)),
        out_shape=jax.ShapeDtypeStruct((N, 1), jnp.float32),
        compiler_params=pltpu.CompilerParams(
            dimension_semantics=("arbitrary",)),
        interpret=interpret,
    )(sc, labn, ones)

    pdT = pred_dist.reshape(N, 2 * _NBINS).T.reshape(2 * _NBINS, NR, _LANES)
    bw2d = bw_n.reshape(NR, _LANES)
    x0p = pred_bboxes[..., 0].reshape(NR, _LANES)
    x1p = pred_bboxes[..., 1].reshape(NR, _LANES)
    ap = anchor_points.reshape(NR, _LANES)
    lab = assigned_labels.reshape(NR, _LANES)
    x0a = assigned_bboxes[..., 0].reshape(NR, _LANES)
    x1a = assigned_bboxes[..., 1].reshape(NR, _LANES)
    ssum = assigned_scores_sum.reshape(1)

    row_spec = pl.BlockSpec((RB, _LANES), lambda i: (i, 0))
    out = pl.pallas_call(
        _loss_body,
        grid=grid,
        in_specs=[
            pl.BlockSpec((2 * _NBINS, RB, _LANES), lambda i: (0, i, 0)),
            row_spec, row_spec, row_spec, row_spec, row_spec, row_spec,
            row_spec,
            pl.BlockSpec(memory_space=pltpu.SMEM),
        ],
        out_specs=[
            pl.BlockSpec(memory_space=pltpu.SMEM),
            pl.BlockSpec(memory_space=pltpu.SMEM),
            pl.BlockSpec(memory_space=pltpu.SMEM),
        ],
        out_shape=[
            jax.ShapeDtypeStruct((1,), jnp.float32),
            jax.ShapeDtypeStruct((1,), jnp.float32),
            jax.ShapeDtypeStruct((1,), jnp.float32),
        ],
        scratch_shapes=[pltpu.SMEM((8,), jnp.float32)],
        compiler_params=pltpu.CompilerParams(
            dimension_semantics=("arbitrary",)),
        interpret=interpret,
    )(pdT, bw2d, x0p, x1p, ap, lab, x0a, x1a, ssum)
    return (out[0][0], out[1][0], out[2][0])


def kernel(pred_dist, pred_bboxes, anchor_points, assigned_labels,
           assigned_bboxes, assigned_scores, assigned_scores_sum):
    return _run(pred_dist, pred_bboxes, anchor_points, assigned_labels,
                assigned_bboxes, assigned_scores, assigned_scores_sum)


# speedup vs baseline: 1.5978x; 1.5978x over previous
import functools

import jax
import jax.numpy as jnp
from jax import lax
from jax.experimental import pallas as pl
from jax.experimental.pallas import tpu as pltpu

_NUM_CLASSES = 80
_REG_MAX = 16
_NBINS = _NB = _REG_MAX + 1
_W = 8192
_LANES = 128
_RB = _W // _LANES


def _fmt_body(sc_ref, pd_ref, rs_ref, pdt_ref):
    sc = sc_ref[...]
    ones8 = jnp.ones((8, _NUM_CLASSES), jnp.float32)
    rs8 = lax.dot_general(ones8, sc, (((1,), (1,)), ((), ())),
                          preferred_element_type=jnp.float32)
    rs_ref[...] = rs8[0:1].reshape(1, _RB, _LANES)
    i0 = lax.broadcasted_iota(jnp.int32, (2 * _NB, 2 * _NB), 0)
    i1 = lax.broadcasted_iota(jnp.int32, (2 * _NB, 2 * _NB), 1)
    eye = (i0 == i1).astype(jnp.float32)
    pdt = lax.dot_general(eye, pd_ref[...], (((1,), (1,)), ((), ())),
                          preferred_element_type=jnp.float32)
    pdt_ref[...] = pdt.reshape(2 * _NB, _RB, _LANES)


def _loss_body(pdt_ref, rs_ref, x0p_ref, x1p_ref, ap_ref, lab_ref, x0a_ref,
               x1a_ref, ssum_ref, l1_ref, iou_ref, dfl_ref, acc_ref):
    step = pl.program_id(0)
    nsteps = pl.num_programs(0)

    @pl.when(step == 0)
    def _init():
        for i in range(5):
            acc_ref[i] = 0.0

    mask = lab_ref[...] != _NUM_CLASSES
    maskf = mask.astype(jnp.float32)

    x0p = x0p_ref[...]
    x1p = x1p_ref[...]
    x0a = x0a_ref[...]
    x1a = x1a_ref[...]
    l1sum = jnp.sum((jnp.abs(x0p - x0a) + jnp.abs(x1p - x1a)) * maskf)

    inter = jnp.maximum(jnp.minimum(x1p, x1a) - jnp.maximum(x0p, x0a), 0.0)
    union = (x1p - x0p) + (x1a - x0a) - inter
    union_safe = jnp.where(mask, union, 1.0)
    tiou = jnp.where(mask, inter / union_safe, 0.0)
    iousum = jnp.sum(jnp.where(mask, 1.0 - tiou, 0.0))
    npos = jnp.sum(maskf)

    bw = rs_ref[0] * maskf
    bwsum = jnp.sum(bw)

    ap = ap_ref[...]
    ltrb_l = jnp.clip(ap - x0a, 0.0, _REG_MAX - 0.01)
    ltrb_r = jnp.clip(x1a - ap, 0.0, _REG_MAX - 0.01)
    pd = pdt_ref[...]
    iota3 = lax.broadcasted_iota(jnp.int32, (_NB,) + ap.shape, 0)

    def _dfl_half(x, ltrb):
        logS = jnp.log(jnp.sum(jnp.exp(x), axis=0))
        t = ltrb.astype(jnp.int32)
        xt = jnp.sum(jnp.where(iota3 == t[None], x, 0.0), axis=0)
        xt1 = jnp.sum(jnp.where(iota3 == t[None] + 1, x, 0.0), axis=0)
        wl = (t + 1).astype(jnp.float32) - ltrb
        wr = 1.0 - wl
        return (logS - xt) * wl + (logS - xt1) * wr

    dfl = 0.5 * (_dfl_half(pd[:_NB], ltrb_l) + _dfl_half(pd[_NB:], ltrb_r))
    dflsum = jnp.sum(dfl * bw)

    acc_ref[0] += npos
    acc_ref[1] += l1sum
    acc_ref[2] += iousum
    acc_ref[3] += bwsum
    acc_ref[4] += dflsum

    @pl.when(step == nsteps - 1)
    def _finish():
        np_ = acc_ref[0]
        ssum = ssum_ref[0]
        l1_ref[0] = acc_ref[1] / (np_ * 2.0)
        iou_ref[0] = (acc_ref[2] / np_) * acc_ref[3] / ssum
        dfl_ref[0] = acc_ref[4] / ssum


@functools.partial(jax.jit, static_argnames=("interpret",))
def _run(pred_dist, pred_bboxes, anchor_points, assigned_labels,
         assigned_bboxes, assigned_scores, assigned_scores_sum,
         interpret=False):
    B, L = assigned_labels.shape
    N = B * L
    NR = N // _LANES
    R = N // _W

    sc = assigned_scores.reshape(N, _NUM_CLASSES)
    pd = pred_dist.reshape(N, 2 * _NB)
    rsT, pdT = pl.pallas_call(
        _fmt_body,
        grid=(R,),
        in_specs=[
            pl.BlockSpec((_W, _NUM_CLASSES), lambda i: (i, 0)),
            pl.BlockSpec((_W, 2 * _NB), lambda i: (i, 0)),
        ],
        out_specs=[
            pl.BlockSpec((1, _RB, _LANES), lambda i: (0, i, 0)),
            pl.BlockSpec((2 * _NB, _RB, _LANES), lambda i: (0, i, 0)),
        ],
        out_shape=[
            jax.ShapeDtypeStruct((1, NR, _LANES), jnp.float32),
            jax.ShapeDtypeStruct((2 * _NB, NR, _LANES), jnp.float32),
        ],
        compiler_params=pltpu.CompilerParams(
            dimension_semantics=("arbitrary",)),
        interpret=interpret,
    )(sc, pd)

    x0p = pred_bboxes[..., 0].reshape(NR, _LANES)
    x1p = pred_bboxes[..., 1].reshape(NR, _LANES)
    ap = anchor_points.reshape(NR, _LANES)
    lab = assigned_labels.reshape(NR, _LANES)
    x0a = assigned_bboxes[..., 0].reshape(NR, _LANES)
    x1a = assigned_bboxes[..., 1].reshape(NR, _LANES)
    ssum = assigned_scores_sum.reshape(1)

    row_spec = pl.BlockSpec((_RB, _LANES), lambda i: (i, 0))
    out = pl.pallas_call(
        _loss_body,
        grid=(R,),
        in_specs=[
            pl.BlockSpec((2 * _NB, _RB, _LANES), lambda i: (0, i, 0)),
            pl.BlockSpec((1, _RB, _LANES), lambda i: (0, i, 0)),
            row_spec, row_spec, row_spec, row_spec, row_spec, row_spec,
            pl.BlockSpec(memory_space=pltpu.SMEM),
        ],
        out_specs=[
            pl.BlockSpec(memory_space=pltpu.SMEM),
            pl.BlockSpec(memory_space=pltpu.SMEM),
            pl.BlockSpec(memory_space=pltpu.SMEM),
        ],
        out_shape=[
            jax.ShapeDtypeStruct((1,), jnp.float32),
            jax.ShapeDtypeStruct((1,), jnp.float32),
            jax.ShapeDtypeStruct((1,), jnp.float32),
        ],
        scratch_shapes=[pltpu.SMEM((8,), jnp.float32)],
        compiler_params=pltpu.CompilerParams(
            dimension_semantics=("arbitrary",)),
        interpret=interpret,
    )(pdT, rsT, x0p, x1p, ap, lab, x0a, x1a, ssum)
    return (out[0][0], out[1][0], out[2][0])


def kernel(pred_dist, pred_bboxes, anchor_points, assigned_labels,
           assigned_bboxes, assigned_scores, assigned_scores_sum):
    return _run(pred_dist, pred_bboxes, anchor_points, assigned_labels,
                assigned_bboxes, assigned_scores, assigned_scores_sum)
